# baseline (speedup 1.0000x reference)
import jax
import jax.numpy as jnp
import numpy as np
from jax import lax
from jax.experimental import pallas as pl
from jax.experimental.pallas import tpu as pltpu
from jax.experimental.pallas import tpu_sc as plsc

_N = 10000
_E = 320000
_NPAD = 10240
_K = 128
_TILES = 16
_CH = 160
_EPAD = _TILES * _CH * _K


def _make_sc_scatter(D):
    Dh = D // 2
    Dw = Dh + 16
    rows_pt = _NPAD // _TILES
    nrep = rows_pt // _K
    gbuf = 4
    sbuf = 2
    nr = _CH // gbuf

    def body(table, ei_h, zrows_h,
             acc_out,
             src_v, dst_v, braw_r, rows_r, acc_sh, gsem, ssem):
        c = lax.axis_index("c")
        s = lax.axis_index("s")
        base = s * rows_pt

        pltpu.sync_copy(zrows_h, rows_r.at[0])
        for r in range(nrep):
            pltpu.sync_copy(rows_r.at[0], acc_sh.at[pl.ds(base + r * _K, _K)])
        ones16 = jnp.full((16,), 1.0, jnp.float32)

        def onesinit(r, carry):
            for b in range(sbuf):
                rows_r[b, r, pl.ds(Dh, 16)] = ones16
            return carry

        lax.fori_loop(0, _K, onesinit, 0)
        pltpu.sync_copy(ei_h.at[0, pl.ds(s * _CH, _CH)], src_v)
        pltpu.sync_copy(ei_h.at[1, pl.ds(s * _CH, _CH)], dst_v)

        def idxfix(r, carry):
            for g2 in range(_K // 16):
                sl = pl.ds(16 * g2, 16)
                src_v[r, sl] = src_v[r, sl] * 2 + c
            return carry

        lax.fori_loop(0, _CH, idxfix, 0)
        plsc.subcore_barrier()

        def convert(b, rb):
            def rowconv(r8, carry):
                for u in range(8):
                    r = r8 * 8 + u
                    for g in range(Dh // 32):
                        w = plsc.bitcast(
                            braw_r[b, r, pl.ds(32 * g, 32)], jnp.uint32)
                        rows_r[rb, r, pl.ds(32 * g, 16)] = plsc.bitcast(
                            w << 16, jnp.float32)
                        rows_r[rb, r, pl.ds(32 * g + 16, 16)] = plsc.bitcast(
                            w & np.uint32(0xFFFF0000), jnp.float32)
                return carry

            lax.fori_loop(0, _K // 8, rowconv, 0)

        for b in range(gbuf):
            pltpu.async_copy(table.at[src_v.at[b]], braw_r.at[b], gsem.at[b])

        def round_(r, carry):
            j0 = r * gbuf
            for b in range(gbuf):
                j = j0 + b
                rb = b % sbuf
                pltpu.make_async_copy(
                    table.at[src_v.at[j]], braw_r.at[b], gsem.at[b]).wait()

                if b < sbuf:
                    @pl.when(r > 0)
                    def _():
                        pltpu.make_async_copy(
                            rows_r.at[rb], acc_sh.at[dst_v.at[j]],
                            ssem.at[rb]).wait()
                else:
                    pltpu.make_async_copy(
                        rows_r.at[rb], acc_sh.at[dst_v.at[j]],
                        ssem.at[rb]).wait()

                convert(b, rb)

                @pl.when(r + 1 < nr)
                def _():
                    pltpu.async_copy(
                        table.at[src_v.at[j + gbuf]], braw_r.at[b],
                        gsem.at[b])

                pltpu.async_copy(
                    rows_r.at[rb], acc_sh.at[dst_v.at[j]], ssem.at[rb],
                    add=True)
            return carry

        lax.fori_loop(0, nr, round_, 0)
        for b in range(gbuf - sbuf, gbuf):
            pltpu.make_async_copy(
                rows_r.at[b % sbuf], acc_sh.at[dst_v.at[(nr - 1) * gbuf + b]],
                ssem.at[b % sbuf]).wait()

        plsc.subcore_barrier()
        pltpu.sync_copy(acc_sh.at[pl.ds(base, rows_pt)],
                        acc_out.at[c, pl.ds(base, rows_pt)])

    return pl.kernel(
        body,
        out_type=jax.ShapeDtypeStruct((2, _NPAD, Dw), jnp.float32),
        mesh=plsc.VectorSubcoreMesh(core_axis_name="c", subcore_axis_name="s"),
        compiler_params=pltpu.CompilerParams(
            use_tc_tiling_on_sc=False, needs_layout_passes=False),
        scratch_types=[
            pltpu.VMEM((_CH, _K), jnp.int32),
            pltpu.VMEM((_CH, _K), jnp.int32),
            pltpu.VMEM((4, _K, Dh), jnp.bfloat16),
            pltpu.VMEM((2, _K, Dw), jnp.float32),
            pltpu.VMEM_SHARED((_NPAD, Dw), jnp.float32),
            pltpu.SemaphoreType.DMA((4,)),
            pltpu.SemaphoreType.DMA((2,)),
        ],
    )


_sc_scatter_128 = _make_sc_scatter(128)
_sc_scatter_64 = _make_sc_scatter(64)


_BLK = 1280


def _row_spec(cols):
    return pl.BlockSpec((_BLK, cols), lambda i: (i, 0))


def _full_spec(rows, cols):
    return pl.BlockSpec((rows, cols), lambda i: (0, 0))


def _mid_body(a0, a1, x, w1r, w1l, b1, w2l, w2r, p2_o, r2_o):
    deg = a0[...][:, 64:65]
    agg = jnp.concatenate([a0[...][:, :64], a1[...][:, :64]], axis=1)
    agg = agg / jnp.maximum(deg, 1.0)
    r1 = jnp.dot(x[...], w1r[...], preferred_element_type=jnp.float32)
    h = agg @ w1l[...] + b1[...] + r1
    h = jnp.maximum(h, 0.0)
    p2_o[...] = jnp.dot(h, w2l[...], preferred_element_type=jnp.float32).astype(jnp.bfloat16)
    r2_o[...] = jnp.dot(h, w2r[...], preferred_element_type=jnp.float32)


def _tc_mid(a0, a1, x, w1r, w1l, b1, w2l, w2r):
    m = a0.shape[0]
    return pl.pallas_call(
        _mid_body,
        grid=(m // _BLK,),
        in_specs=[_row_spec(80), _row_spec(80), _row_spec(128),
                  _full_spec(128, 128), _full_spec(128, 128),
                  _full_spec(1, 128),
                  _full_spec(128, 64), _full_spec(128, 64)],
        out_specs=(_row_spec(64), _row_spec(64)),
        out_shape=(jax.ShapeDtypeStruct((m, 64), jnp.bfloat16),
                   jax.ShapeDtypeStruct((m, 64), jnp.float32)),
    )(a0, a1, x, w1r, w1l, b1, w2l, w2r)


def _post_body(q0, q1, r2, b2, o_ref):
    deg = q0[...][:, 32:33]
    agg = jnp.concatenate([q0[...][:, :32], q1[...][:, :32]], axis=1)
    agg = agg / jnp.maximum(deg, 1.0)
    o_ref[...] = agg + b2[...] + r2[...]


def _tc_post(q0, q1, r2, b2):
    m = q0.shape[0]
    return pl.pallas_call(
        _post_body,
        grid=(m // _BLK,),
        in_specs=[_row_spec(48), _row_spec(48), _row_spec(64),
                  _full_spec(1, 64)],
        out_specs=_row_spec(64),
        out_shape=jax.ShapeDtypeStruct((m, 64), jnp.float32),
    )(q0, q1, r2, b2)


def _prep_edges(edge_index):
    return jnp.pad(edge_index, ((0, 0), (0, _EPAD - _E)),
                   constant_values=_N).reshape(2, _TILES * _CH, _K)


_G32 = np.array([2 * r for r in range(16)] +
                [2 * r + 1 for r in range(16)])
_G32INV = np.argsort(_G32)
_GAMMA128 = np.concatenate([64 * c + 32 * g + _G32
                            for c in range(2) for g in range(2)])
_COLPERM64 = np.concatenate([32 * c + _G32INV for c in range(2)])


def kernel(x, edge_index1, edge_index2, W1_l, b1, W1_r, W2_l, b2, W2_r):
    xpad = jnp.pad(x, ((0, _NPAD - _N), (0, 0)))
    ei1 = _prep_edges(edge_index1)
    ei2 = _prep_edges(edge_index2)
    z80 = jnp.zeros((_K, 80), jnp.float32)
    z48 = jnp.zeros((_K, 48), jnp.float32)

    xbf = xpad.astype(jnp.bfloat16).reshape(2 * _NPAD, 64)
    acc1 = _sc_scatter_128(xbf, ei1, z80)
    p2, r2 = _tc_mid(acc1[0], acc1[1], xpad, W1_r,
                     W1_l[_GAMMA128, :], b1.reshape(1, 128),
                     W2_l[:, _COLPERM64], W2_r)
    acc2 = _sc_scatter_64(p2.reshape(2 * _NPAD, 32), ei2, z48)
    out = _tc_post(acc2[0], acc2[1], r2, b2.reshape(1, 64))
    return out[:_N]


# speedup vs baseline: 1.0030x; 1.0030x over previous
import jax
import jax.numpy as jnp
import numpy as np
from jax import lax
from jax.experimental import pallas as pl
from jax.experimental.pallas import tpu as pltpu
from jax.experimental.pallas import tpu_sc as plsc

_N = 10000
_E = 320000
_NPAD = 10240
_K = 128
_TILES = 16
_CH = 160
_EPAD = _TILES * _CH * _K


def _make_sc_scatter(D):
    Dh = D // 2
    Dw = Dh + 16
    rows_pt = _NPAD // _TILES
    nrep = rows_pt // _K
    gbuf = 4 if D == 128 else 8
    sbuf = 2
    nr = _CH // gbuf

    def body(table, ei_h, zrows_h,
             acc_out,
             src_v, dst_v, braw_r, rows_r, acc_sh, gsem, ssem):
        c = lax.axis_index("c")
        s = lax.axis_index("s")
        base = s * rows_pt

        pltpu.sync_copy(zrows_h, rows_r.at[0])
        for r in range(nrep):
            pltpu.sync_copy(rows_r.at[0], acc_sh.at[pl.ds(base + r * _K, _K)])
        ones16 = jnp.full((16,), 1.0, jnp.float32)

        def onesinit(r, carry):
            for b in range(sbuf):
                rows_r[b, r, pl.ds(Dh, 16)] = ones16
            return carry

        lax.fori_loop(0, _K, onesinit, 0)
        pltpu.sync_copy(ei_h.at[0, pl.ds(s * _CH, _CH)], src_v)
        pltpu.sync_copy(ei_h.at[1, pl.ds(s * _CH, _CH)], dst_v)

        def idxfix(r, carry):
            for g2 in range(_K // 16):
                sl = pl.ds(16 * g2, 16)
                src_v[r, sl] = src_v[r, sl] * 2 + c
            return carry

        lax.fori_loop(0, _CH, idxfix, 0)
        plsc.subcore_barrier()

        def convert(b, rb):
            def rowconv(r8, carry):
                for u in range(8):
                    r = r8 * 8 + u
                    for g in range(Dh // 32):
                        w = plsc.bitcast(
                            braw_r[b, r, pl.ds(32 * g, 32)], jnp.uint32)
                        rows_r[rb, r, pl.ds(32 * g, 16)] = plsc.bitcast(
                            w << 16, jnp.float32)
                        rows_r[rb, r, pl.ds(32 * g + 16, 16)] = plsc.bitcast(
                            w & np.uint32(0xFFFF0000), jnp.float32)
                return carry

            lax.fori_loop(0, _K // 8, rowconv, 0)

        for b in range(gbuf):
            pltpu.async_copy(table.at[src_v.at[b]], braw_r.at[b], gsem.at[b])

        def round_(r, carry):
            j0 = r * gbuf
            for b in range(gbuf):
                j = j0 + b
                rb = b % sbuf
                pltpu.make_async_copy(
                    table.at[src_v.at[j]], braw_r.at[b], gsem.at[b]).wait()

                if b < sbuf:
                    @pl.when(r > 0)
                    def _():
                        pltpu.make_async_copy(
                            rows_r.at[rb], acc_sh.at[dst_v.at[j]],
                            ssem.at[rb]).wait()
                else:
                    pltpu.make_async_copy(
                        rows_r.at[rb], acc_sh.at[dst_v.at[j]],
                        ssem.at[rb]).wait()

                convert(b, rb)

                @pl.when(r + 1 < nr)
                def _():
                    pltpu.async_copy(
                        table.at[src_v.at[j + gbuf]], braw_r.at[b],
                        gsem.at[b])

                pltpu.async_copy(
                    rows_r.at[rb], acc_sh.at[dst_v.at[j]], ssem.at[rb],
                    add=True)
            return carry

        lax.fori_loop(0, nr, round_, 0)
        for b in range(gbuf - sbuf, gbuf):
            pltpu.make_async_copy(
                rows_r.at[b % sbuf], acc_sh.at[dst_v.at[(nr - 1) * gbuf + b]],
                ssem.at[b % sbuf]).wait()

        plsc.subcore_barrier()
        pltpu.sync_copy(acc_sh.at[pl.ds(base, rows_pt)],
                        acc_out.at[c, pl.ds(base, rows_pt)])

    return pl.kernel(
        body,
        out_type=jax.ShapeDtypeStruct((2, _NPAD, Dw), jnp.float32),
        mesh=plsc.VectorSubcoreMesh(core_axis_name="c", subcore_axis_name="s"),
        compiler_params=pltpu.CompilerParams(
            use_tc_tiling_on_sc=False, needs_layout_passes=False),
        scratch_types=[
            pltpu.VMEM((_CH, _K), jnp.int32),
            pltpu.VMEM((_CH, _K), jnp.int32),
            pltpu.VMEM((gbuf, _K, Dh), jnp.bfloat16),
            pltpu.VMEM((2, _K, Dw), jnp.float32),
            pltpu.VMEM_SHARED((_NPAD, Dw), jnp.float32),
            pltpu.SemaphoreType.DMA((gbuf,)),
            pltpu.SemaphoreType.DMA((2,)),
        ],
    )


_sc_scatter_128 = _make_sc_scatter(128)
_sc_scatter_64 = _make_sc_scatter(64)


_BLK = 1280


def _row_spec(cols):
    return pl.BlockSpec((_BLK, cols), lambda i: (i, 0))


def _full_spec(rows, cols):
    return pl.BlockSpec((rows, cols), lambda i: (0, 0))


def _mid_body(a0, a1, x, w1r, w1l, b1, w2l, w2r, p2_o, r2_o):
    deg = a0[...][:, 64:65]
    agg = jnp.concatenate([a0[...][:, :64], a1[...][:, :64]], axis=1)
    agg = agg / jnp.maximum(deg, 1.0)
    r1 = jnp.dot(x[...], w1r[...], preferred_element_type=jnp.float32)
    h = agg @ w1l[...] + b1[...] + r1
    h = jnp.maximum(h, 0.0)
    p2_o[...] = jnp.dot(h, w2l[...], preferred_element_type=jnp.float32).astype(jnp.bfloat16)
    r2_o[...] = jnp.dot(h, w2r[...], preferred_element_type=jnp.float32)


def _tc_mid(a0, a1, x, w1r, w1l, b1, w2l, w2r):
    m = a0.shape[0]
    return pl.pallas_call(
        _mid_body,
        grid=(m // _BLK,),
        in_specs=[_row_spec(80), _row_spec(80), _row_spec(128),
                  _full_spec(128, 128), _full_spec(128, 128),
                  _full_spec(1, 128),
                  _full_spec(128, 64), _full_spec(128, 64)],
        out_specs=(_row_spec(64), _row_spec(64)),
        out_shape=(jax.ShapeDtypeStruct((m, 64), jnp.bfloat16),
                   jax.ShapeDtypeStruct((m, 64), jnp.float32)),
    )(a0, a1, x, w1r, w1l, b1, w2l, w2r)


def _post_body(q0, q1, r2, b2, o_ref):
    deg = q0[...][:, 32:33]
    agg = jnp.concatenate([q0[...][:, :32], q1[...][:, :32]], axis=1)
    agg = agg / jnp.maximum(deg, 1.0)
    o_ref[...] = agg + b2[...] + r2[...]


def _tc_post(q0, q1, r2, b2):
    m = q0.shape[0]
    return pl.pallas_call(
        _post_body,
        grid=(m // _BLK,),
        in_specs=[_row_spec(48), _row_spec(48), _row_spec(64),
                  _full_spec(1, 64)],
        out_specs=_row_spec(64),
        out_shape=jax.ShapeDtypeStruct((m, 64), jnp.float32),
    )(q0, q1, r2, b2)


def _prep_edges(edge_index):
    return jnp.pad(edge_index, ((0, 0), (0, _EPAD - _E)),
                   constant_values=_N).reshape(2, _TILES * _CH, _K)


_G32 = np.array([2 * r for r in range(16)] +
                [2 * r + 1 for r in range(16)])
_G32INV = np.argsort(_G32)
_GAMMA128 = np.concatenate([64 * c + 32 * g + _G32
                            for c in range(2) for g in range(2)])
_COLPERM64 = np.concatenate([32 * c + _G32INV for c in range(2)])


def kernel(x, edge_index1, edge_index2, W1_l, b1, W1_r, W2_l, b2, W2_r):
    xpad = jnp.pad(x, ((0, _NPAD - _N), (0, 0)))
    ei1 = _prep_edges(edge_index1)
    ei2 = _prep_edges(edge_index2)
    z80 = jnp.zeros((_K, 80), jnp.float32)
    z48 = jnp.zeros((_K, 48), jnp.float32)

    xbf = xpad.astype(jnp.bfloat16).reshape(2 * _NPAD, 64)
    acc1 = _sc_scatter_128(xbf, ei1, z80)
    p2, r2 = _tc_mid(acc1[0], acc1[1], xpad, W1_r,
                     W1_l[_GAMMA128, :], b1.reshape(1, 128),
                     W2_l[:, _COLPERM64], W2_r)
    acc2 = _sc_scatter_64(p2.reshape(2 * _NPAD, 32), ei2, z48)
    out = _tc_post(acc2[0], acc2[1], r2, b2.reshape(1, 64))
    return out[:_N]


# speedup vs baseline: 1.0035x; 1.0004x over previous
import jax
import jax.numpy as jnp
import numpy as np
from jax import lax
from jax.experimental import pallas as pl
from jax.experimental.pallas import tpu as pltpu
from jax.experimental.pallas import tpu_sc as plsc

_N = 10000
_E = 320000
_NPAD = 10240
_K = 128
_TILES = 16
_CH = 160
_EPAD = _TILES * _CH * _K


def _make_sc_scatter(D):
    Dh = D // 2
    Dw = Dh + 16
    rows_pt = _NPAD // _TILES
    nrep = rows_pt // _K
    gbuf = 4 if D == 128 else 8
    sbuf = 2 if D == 128 else 4
    nr = _CH // gbuf

    def body(table, ei_h, zrows_h,
             acc_out,
             src_v, dst_v, braw_r, rows_r, acc_sh, gsem, ssem):
        c = lax.axis_index("c")
        s = lax.axis_index("s")
        base = s * rows_pt

        pltpu.sync_copy(zrows_h, rows_r.at[0])
        for r in range(nrep):
            pltpu.sync_copy(rows_r.at[0], acc_sh.at[pl.ds(base + r * _K, _K)])
        ones16 = jnp.full((16,), 1.0, jnp.float32)

        def onesinit(r, carry):
            for b in range(sbuf):
                rows_r[b, r, pl.ds(Dh, 16)] = ones16
            return carry

        lax.fori_loop(0, _K, onesinit, 0)
        pltpu.sync_copy(ei_h.at[0, pl.ds(s * _CH, _CH)], src_v)
        pltpu.sync_copy(ei_h.at[1, pl.ds(s * _CH, _CH)], dst_v)

        def idxfix(r, carry):
            for g2 in range(_K // 16):
                sl = pl.ds(16 * g2, 16)
                src_v[r, sl] = src_v[r, sl] * 2 + c
            return carry

        lax.fori_loop(0, _CH, idxfix, 0)
        plsc.subcore_barrier()

        def convert(b, rb):
            def rowconv(r8, carry):
                for u in range(8):
                    r = r8 * 8 + u
                    for g in range(Dh // 32):
                        w = plsc.bitcast(
                            braw_r[b, r, pl.ds(32 * g, 32)], jnp.uint32)
                        rows_r[rb, r, pl.ds(32 * g, 16)] = plsc.bitcast(
                            w << 16, jnp.float32)
                        rows_r[rb, r, pl.ds(32 * g + 16, 16)] = plsc.bitcast(
                            w & np.uint32(0xFFFF0000), jnp.float32)
                return carry

            lax.fori_loop(0, _K // 8, rowconv, 0)

        for b in range(gbuf):
            pltpu.async_copy(table.at[src_v.at[b]], braw_r.at[b], gsem.at[b])

        def round_(r, carry):
            j0 = r * gbuf
            for b in range(gbuf):
                j = j0 + b
                rb = b % sbuf
                pltpu.make_async_copy(
                    table.at[src_v.at[j]], braw_r.at[b], gsem.at[b]).wait()

                if b < sbuf:
                    @pl.when(r > 0)
                    def _():
                        pltpu.make_async_copy(
                            rows_r.at[rb], acc_sh.at[dst_v.at[j]],
                            ssem.at[rb]).wait()
                else:
                    pltpu.make_async_copy(
                        rows_r.at[rb], acc_sh.at[dst_v.at[j]],
                        ssem.at[rb]).wait()

                convert(b, rb)

                @pl.when(r + 1 < nr)
                def _():
                    pltpu.async_copy(
                        table.at[src_v.at[j + gbuf]], braw_r.at[b],
                        gsem.at[b])

                pltpu.async_copy(
                    rows_r.at[rb], acc_sh.at[dst_v.at[j]], ssem.at[rb],
                    add=True)
            return carry

        lax.fori_loop(0, nr, round_, 0)
        for b in range(gbuf - sbuf, gbuf):
            pltpu.make_async_copy(
                rows_r.at[b % sbuf], acc_sh.at[dst_v.at[(nr - 1) * gbuf + b]],
                ssem.at[b % sbuf]).wait()

        plsc.subcore_barrier()
        pltpu.sync_copy(acc_sh.at[pl.ds(base, rows_pt)],
                        acc_out.at[c, pl.ds(base, rows_pt)])

    return pl.kernel(
        body,
        out_type=jax.ShapeDtypeStruct((2, _NPAD, Dw), jnp.float32),
        mesh=plsc.VectorSubcoreMesh(core_axis_name="c", subcore_axis_name="s"),
        compiler_params=pltpu.CompilerParams(
            use_tc_tiling_on_sc=False, needs_layout_passes=False),
        scratch_types=[
            pltpu.VMEM((_CH, _K), jnp.int32),
            pltpu.VMEM((_CH, _K), jnp.int32),
            pltpu.VMEM((gbuf, _K, Dh), jnp.bfloat16),
            pltpu.VMEM((sbuf, _K, Dw), jnp.float32),
            pltpu.VMEM_SHARED((_NPAD, Dw), jnp.float32),
            pltpu.SemaphoreType.DMA((gbuf,)),
            pltpu.SemaphoreType.DMA((sbuf,)),
        ],
    )


_sc_scatter_128 = _make_sc_scatter(128)
_sc_scatter_64 = _make_sc_scatter(64)


_BLK = 1280


def _row_spec(cols):
    return pl.BlockSpec((_BLK, cols), lambda i: (i, 0))


def _full_spec(rows, cols):
    return pl.BlockSpec((rows, cols), lambda i: (0, 0))


def _mid_body(a0, a1, x, w1r, w1l, b1, w2l, w2r, p2_o, r2_o):
    deg = a0[...][:, 64:65]
    agg = jnp.concatenate([a0[...][:, :64], a1[...][:, :64]], axis=1)
    agg = agg / jnp.maximum(deg, 1.0)
    r1 = jnp.dot(x[...], w1r[...], preferred_element_type=jnp.float32)
    h = agg @ w1l[...] + b1[...] + r1
    h = jnp.maximum(h, 0.0)
    p2_o[...] = jnp.dot(h, w2l[...], preferred_element_type=jnp.float32).astype(jnp.bfloat16)
    r2_o[...] = jnp.dot(h, w2r[...], preferred_element_type=jnp.float32)


def _tc_mid(a0, a1, x, w1r, w1l, b1, w2l, w2r):
    m = a0.shape[0]
    return pl.pallas_call(
        _mid_body,
        grid=(m // _BLK,),
        in_specs=[_row_spec(80), _row_spec(80), _row_spec(128),
                  _full_spec(128, 128), _full_spec(128, 128),
                  _full_spec(1, 128),
                  _full_spec(128, 64), _full_spec(128, 64)],
        out_specs=(_row_spec(64), _row_spec(64)),
        out_shape=(jax.ShapeDtypeStruct((m, 64), jnp.bfloat16),
                   jax.ShapeDtypeStruct((m, 64), jnp.float32)),
    )(a0, a1, x, w1r, w1l, b1, w2l, w2r)


def _post_body(q0, q1, r2, b2, o_ref):
    deg = q0[...][:, 32:33]
    agg = jnp.concatenate([q0[...][:, :32], q1[...][:, :32]], axis=1)
    agg = agg / jnp.maximum(deg, 1.0)
    o_ref[...] = agg + b2[...] + r2[...]


def _tc_post(q0, q1, r2, b2):
    m = q0.shape[0]
    return pl.pallas_call(
        _post_body,
        grid=(m // _BLK,),
        in_specs=[_row_spec(48), _row_spec(48), _row_spec(64),
                  _full_spec(1, 64)],
        out_specs=_row_spec(64),
        out_shape=jax.ShapeDtypeStruct((m, 64), jnp.float32),
    )(q0, q1, r2, b2)


def _prep_edges(edge_index):
    return jnp.pad(edge_index, ((0, 0), (0, _EPAD - _E)),
                   constant_values=_N).reshape(2, _TILES * _CH, _K)


_G32 = np.array([2 * r for r in range(16)] +
                [2 * r + 1 for r in range(16)])
_G32INV = np.argsort(_G32)
_GAMMA128 = np.concatenate([64 * c + 32 * g + _G32
                            for c in range(2) for g in range(2)])
_COLPERM64 = np.concatenate([32 * c + _G32INV for c in range(2)])


def kernel(x, edge_index1, edge_index2, W1_l, b1, W1_r, W2_l, b2, W2_r):
    xpad = jnp.pad(x, ((0, _NPAD - _N), (0, 0)))
    ei1 = _prep_edges(edge_index1)
    ei2 = _prep_edges(edge_index2)
    z80 = jnp.zeros((_K, 80), jnp.float32)
    z48 = jnp.zeros((_K, 48), jnp.float32)

    xbf = xpad.astype(jnp.bfloat16).reshape(2 * _NPAD, 64)
    acc1 = _sc_scatter_128(xbf, ei1, z80)
    p2, r2 = _tc_mid(acc1[0], acc1[1], xpad, W1_r,
                     W1_l[_GAMMA128, :], b1.reshape(1, 128),
                     W2_l[:, _COLPERM64], W2_r)
    acc2 = _sc_scatter_64(p2.reshape(2 * _NPAD, 32), ei2, z48)
    out = _tc_post(acc2[0], acc2[1], r2, b2.reshape(1, 64))
    return out[:_N]
